# Initial kernel scaffold; baseline (speedup 1.0000x reference)
import jax
import jax.numpy as jnp
from jax.experimental import pallas as pl


def kernel(indexes, codebook):
    raise NotImplementedError("write your pallas kernel here")


# speedup vs baseline: 5.3768x; 5.3768x over previous
import functools

import jax
import jax.numpy as jnp
from jax import lax
from jax.experimental import pallas as pl
from jax.experimental.pallas import tpu as pltpu
from jax.experimental.pallas import tpu_sc as plsc

_info = plsc.get_sparse_core_info()
_NC, _NS = _info.num_cores, _info.num_subcores
_NW = _NC * _NS

_CHUNK = 1024


def kernel(indexes, codebook):
    (B,) = indexes.shape
    V, D = codebook.shape
    b_per_w = B // _NW
    steps = b_per_w // _CHUNK
    mesh = plsc.VectorSubcoreMesh(core_axis_name="c", subcore_axis_name="s")

    @functools.partial(
        pl.kernel,
        mesh=mesh,
        out_type=jax.ShapeDtypeStruct((B, D), jnp.float32),
        compiler_params=pltpu.CompilerParams(use_tc_tiling_on_sc=False),
        scratch_types=[
            pltpu.VMEM((_CHUNK,), jnp.int32),
            pltpu.VMEM((_CHUNK, D), jnp.float32),
            pltpu.SemaphoreType.DMA,
        ],
    )
    def gather_kernel(idx_hbm, table_hbm, out_hbm, idx_v, rows_v, sem):
        wid = lax.axis_index("s") * _NC + lax.axis_index("c")
        base = wid * b_per_w

        def body(g, carry):
            off = base + g * _CHUNK
            pltpu.sync_copy(idx_hbm.at[pl.ds(off, _CHUNK)], idx_v)
            pltpu.async_copy(table_hbm.at[idx_v], rows_v, sem).wait()
            pltpu.sync_copy(rows_v, out_hbm.at[pl.ds(off, _CHUNK)])
            return carry

        lax.fori_loop(0, steps, body, 0)

    return gather_kernel(indexes.astype(jnp.int32), codebook)


# speedup vs baseline: 5.4840x; 1.0199x over previous
import functools

import jax
import jax.numpy as jnp
from jax import lax
from jax.experimental import pallas as pl
from jax.experimental.pallas import tpu as pltpu
from jax.experimental.pallas import tpu_sc as plsc

_info = plsc.get_sparse_core_info()
_NC, _NS = _info.num_cores, _info.num_subcores
_NW = _NC * _NS

_CHUNK = 512
_NBUF = 2


def kernel(indexes, codebook):
    (B,) = indexes.shape
    V, D = codebook.shape
    b_per_w = B // _NW
    steps = b_per_w // _CHUNK
    blocks = steps // _NBUF
    mesh = plsc.VectorSubcoreMesh(core_axis_name="c", subcore_axis_name="s")

    @functools.partial(
        pl.kernel,
        mesh=mesh,
        out_type=jax.ShapeDtypeStruct((B, D), jnp.float32),
        compiler_params=pltpu.CompilerParams(use_tc_tiling_on_sc=False),
        scratch_types=[
            pltpu.VMEM((_CHUNK,), jnp.int32),
            pltpu.VMEM((_CHUNK,), jnp.int32),
            pltpu.VMEM((_CHUNK, D), jnp.float32),
            pltpu.VMEM((_CHUNK, D), jnp.float32),
            pltpu.SemaphoreType.DMA,
            pltpu.SemaphoreType.DMA,
            pltpu.SemaphoreType.DMA,
            pltpu.SemaphoreType.DMA,
            pltpu.SemaphoreType.DMA,
            pltpu.SemaphoreType.DMA,
        ],
    )
    def gather_kernel(idx_hbm, table_hbm, out_hbm,
                      idx0, idx1, rows0, rows1,
                      si0, si1, sg0, sg1, ss0, ss1):
        idx = (idx0, idx1)
        rows = (rows0, rows1)
        si = (si0, si1)
        sg = (sg0, sg1)
        ss = (ss0, ss1)
        wid = lax.axis_index("s") * _NC + lax.axis_index("c")
        base = wid * b_per_w

        for b in range(_NBUF):
            pltpu.async_copy(idx_hbm.at[pl.ds(base + b * _CHUNK, _CHUNK)],
                             idx[b], si[b])

        def block(k, carry):
            for b in range(_NBUF):
                g = k * _NBUF + b
                off = base + g * _CHUNK
                pltpu.make_async_copy(idx_hbm.at[pl.ds(0, _CHUNK)],
                                      idx[b], si[b]).wait()

                @pl.when(k >= 1)
                def _():
                    pltpu.make_async_copy(out_hbm.at[pl.ds(0, _CHUNK)],
                                          rows[b], ss[b]).wait()

                pltpu.async_copy(table_hbm.at[idx[b]], rows[b], sg[b]).wait()

                @pl.when(k < blocks - 1)
                def _():
                    pltpu.async_copy(
                        idx_hbm.at[pl.ds(off + _NBUF * _CHUNK, _CHUNK)],
                        idx[b], si[b])

                pltpu.async_copy(rows[b], out_hbm.at[pl.ds(off, _CHUNK)], ss[b])
            return carry

        lax.fori_loop(0, blocks, block, 0)

        for b in range(_NBUF):
            pltpu.make_async_copy(out_hbm.at[pl.ds(0, _CHUNK)],
                                  rows[b], ss[b]).wait()

    return gather_kernel(indexes.astype(jnp.int32), codebook)


# speedup vs baseline: 6.2346x; 1.1369x over previous
import functools

import jax
import jax.numpy as jnp
from jax import lax
from jax.experimental import pallas as pl
from jax.experimental.pallas import tpu as pltpu
from jax.experimental.pallas import tpu_sc as plsc

_info = plsc.get_sparse_core_info()
_NC, _NS = _info.num_cores, _info.num_subcores
_NW = _NC * _NS

_CHUNK = 512
_NBUF = 2


def kernel(indexes, codebook):
    (B,) = indexes.shape
    V, D = codebook.shape
    b_per_w = B // _NW
    steps = b_per_w // _CHUNK
    blocks = steps // _NBUF
    mesh = plsc.VectorSubcoreMesh(core_axis_name="c", subcore_axis_name="s")

    @functools.partial(
        pl.kernel,
        mesh=mesh,
        out_type=jax.ShapeDtypeStruct((B, D), jnp.float32),
        compiler_params=pltpu.CompilerParams(use_tc_tiling_on_sc=False),
        scratch_types=[
            pltpu.VMEM((_CHUNK,), jnp.int32),
            pltpu.VMEM((_CHUNK,), jnp.int32),
            pltpu.VMEM((_CHUNK, D), jnp.float32),
            pltpu.VMEM((_CHUNK, D), jnp.float32),
            pltpu.SemaphoreType.DMA,
            pltpu.SemaphoreType.DMA,
            pltpu.SemaphoreType.DMA,
            pltpu.SemaphoreType.DMA,
            pltpu.SemaphoreType.DMA,
            pltpu.SemaphoreType.DMA,
            pltpu.VMEM_SHARED((V, D), jnp.float32),
        ],
    )
    def gather_kernel(idx_hbm, table_hbm, out_hbm,
                      idx0, idx1, rows0, rows1,
                      si0, si1, sg0, sg1, ss0, ss1, table_sp):
        idx = (idx0, idx1)
        rows = (rows0, rows1)
        si = (si0, si1)
        sg = (sg0, sg1)
        ss = (ss0, ss1)
        sid = lax.axis_index("s")
        wid = sid * _NC + lax.axis_index("c")
        base = wid * b_per_w

        v_per_s = V // _NS
        pltpu.sync_copy(table_hbm.at[pl.ds(sid * v_per_s, v_per_s)],
                        table_sp.at[pl.ds(sid * v_per_s, v_per_s)])
        plsc.subcore_barrier()

        for b in range(_NBUF):
            pltpu.async_copy(idx_hbm.at[pl.ds(base + b * _CHUNK, _CHUNK)],
                             idx[b], si[b])

        def block(k, carry):
            for b in range(_NBUF):
                g = k * _NBUF + b
                off = base + g * _CHUNK
                pltpu.make_async_copy(idx_hbm.at[pl.ds(0, _CHUNK)],
                                      idx[b], si[b]).wait()

                @pl.when(k >= 1)
                def _():
                    pltpu.make_async_copy(out_hbm.at[pl.ds(0, _CHUNK)],
                                          rows[b], ss[b]).wait()

                pltpu.async_copy(table_sp.at[idx[b]], rows[b], sg[b]).wait()

                @pl.when(k < blocks - 1)
                def _():
                    pltpu.async_copy(
                        idx_hbm.at[pl.ds(off + _NBUF * _CHUNK, _CHUNK)],
                        idx[b], si[b])

                pltpu.async_copy(rows[b], out_hbm.at[pl.ds(off, _CHUNK)], ss[b])
            return carry

        lax.fori_loop(0, blocks, block, 0)

        for b in range(_NBUF):
            pltpu.make_async_copy(out_hbm.at[pl.ds(0, _CHUNK)],
                                  rows[b], ss[b]).wait()

    return gather_kernel(indexes.astype(jnp.int32), codebook)
